# baseline (speedup 1.0000x reference)
import functools

import jax
import jax.numpy as jnp
from jax import lax
from jax.experimental import pallas as pl
from jax.experimental.pallas import tpu as pltpu
from jax.experimental.pallas import tpu_sc as plsc

N = 10000
F_IN = 128
F_HID = 128
F_OUT = 64
E = 320000

NC = 2
NS = 16
NW = NC * NS
CHUNK = 128
K = 80
K2 = 160
E_PAD = NW * K * CHUNK
NPAD = 10240
STRIPE = NPAD // NS
NBUF = 4

_mesh = plsc.VectorSubcoreMesh(core_axis_name="c", subcore_axis_name="s")
_sc_params = pltpu.CompilerParams(use_tc_tiling_on_sc=False)


@functools.partial(
    pl.kernel,
    out_type=jax.ShapeDtypeStruct((NC, NPAD), jnp.float32),
    mesh=_mesh,
    compiler_params=_sc_params,
    scratch_types=[
        pltpu.VMEM((K, CHUNK), jnp.int32),
        pltpu.VMEM((CHUNK,), jnp.float32),
        pltpu.VMEM((STRIPE,), jnp.float32),
        pltpu.VMEM_SHARED((NPAD,), jnp.float32),
        pltpu.SemaphoreType.DMA,
    ],
)
def _deg_kernel(dst_hbm, deg_out, idx_v, ones_v, z_v, deg_sh, sem):
    c = lax.axis_index("c")
    s = lax.axis_index("s")
    w = s * NC + c

    zero16 = jnp.zeros((16,), jnp.float32)
    one16 = jnp.ones((16,), jnp.float32)

    def fill_z(i, carry):
        z_v[pl.ds(i * 16, 16)] = zero16
        return carry

    lax.fori_loop(0, STRIPE // 16, fill_z, 0)

    def fill_o(i, carry):
        ones_v[pl.ds(i * 16, 16)] = one16
        return carry

    lax.fori_loop(0, CHUNK // 16, fill_o, 0)

    pltpu.sync_copy(z_v, deg_sh.at[pl.ds(s * STRIPE, STRIPE)])
    pltpu.sync_copy(dst_hbm.at[w], idx_v)
    plsc.subcore_barrier()

    GRP = 8

    def group(g, carry):
        for b in range(GRP):
            pltpu.async_copy(
                ones_v, deg_sh.at[idx_v.at[g * GRP + b]], sem, add=True)
        for b in range(GRP):
            pltpu.make_async_copy(
                ones_v, deg_sh.at[idx_v.at[g * GRP + b]], sem).wait()
        return carry

    lax.fori_loop(0, K // GRP, group, 0)
    plsc.subcore_barrier()
    pltpu.sync_copy(deg_sh.at[pl.ds(s * STRIPE, STRIPE)],
                    deg_out.at[c, pl.ds(s * STRIPE, STRIPE)])


AGG_NBUF = 5
DEPTH = 2


def _make_agg(F, colsplit, KC):
    scratch = [
        pltpu.VMEM((KC, CHUNK), jnp.int32),
        pltpu.VMEM((KC, CHUNK), jnp.int32),
        pltpu.VMEM((AGG_NBUF, CHUNK, F), jnp.float32),
        pltpu.VMEM_SHARED((NPAD, F), jnp.float32),
    ] + [pltpu.SemaphoreType.DMA] * (2 * AGG_NBUF)

    @functools.partial(
        pl.kernel,
        out_type=jax.ShapeDtypeStruct((NC, NPAD, F), jnp.float32),
        mesh=_mesh,
        compiler_params=_sc_params,
        scratch_types=scratch,
    )
    def _agg(p_hbm, src_hbm, dst_hbm, out, si_v, di_v, bufs, agg_sh, *sems):
        gsems = sems[:AGG_NBUF]
        ssems = sems[AGG_NBUF:]
        c = lax.axis_index("c")
        s = lax.axis_index("s")

        zero16 = jnp.zeros((16,), jnp.float32)

        def zrow(i, carry):
            for j in range(F // 16):
                bufs[0, i, pl.ds(j * 16, 16)] = zero16
            return carry

        lax.fori_loop(0, CHUNK, zrow, 0)
        for i in range(STRIPE // CHUNK):
            pltpu.sync_copy(
                bufs.at[0], agg_sh.at[pl.ds(s * STRIPE + i * CHUNK, CHUNK)])
        if colsplit:
            pltpu.sync_copy(src_hbm.at[c, s], si_v)
            pltpu.sync_copy(dst_hbm.at[s], di_v)
        else:
            w = s * NC + c
            pltpu.sync_copy(src_hbm.at[w], si_v)
            pltpu.sync_copy(dst_hbm.at[w], di_v)
        plsc.subcore_barrier()

        def gather(jj, b):
            pltpu.async_copy(p_hbm.at[si_v.at[jj]], bufs.at[b], gsems[b])

        def gather_wait(jj, b):
            pltpu.make_async_copy(
                p_hbm.at[si_v.at[jj]], bufs.at[b], gsems[b]).wait()

        def scat(jj, b):
            pltpu.async_copy(bufs.at[b], agg_sh.at[di_v.at[jj]], ssems[b],
                             add=True)

        def scat_wait(jj, b):
            pltpu.make_async_copy(
                bufs.at[b], agg_sh.at[di_v.at[jj]], ssems[b]).wait()

        for b in range(DEPTH):
            gather(b, b)

        def step(g, carry):
            for i in range(AGG_NBUF):
                jj = g * AGG_NBUF + i
                bb_r = (i + AGG_NBUF - DEPTH) % AGG_NBUF
                bb_g = (i + DEPTH) % AGG_NBUF

                @pl.when(jj >= DEPTH)
                def _():
                    scat_wait(jj - DEPTH, bb_r)

                @pl.when(jj + DEPTH < KC)
                def _():
                    gather(jj + DEPTH, bb_g)

                gather_wait(jj, i)
                scat(jj, i)
            return carry

        lax.fori_loop(0, KC // AGG_NBUF, step, 0)
        for i in range(DEPTH):
            jj = KC - DEPTH + i
            scat_wait(jj, jj % AGG_NBUF)
        plsc.subcore_barrier()
        pltpu.sync_copy(agg_sh.at[pl.ds(s * STRIPE, STRIPE)],
                        out.at[c, pl.ds(s * STRIPE, STRIPE)])

    return _agg


_agg128 = _make_agg(64, True, K2)
_agg64 = _make_agg(F_OUT, False, K)


BN = 1000


def _tc1(x, deg2, W0):
    def body(x_ref, da_ref, db_ref, w_ref, p_ref, n_ref):
        deg = da_ref[0] + db_ref[0]
        norm = jnp.where(deg > 0, lax.rsqrt(jnp.maximum(deg, 1.0)), 0.0)
        n_ref[...] = norm
        p_ref[...] = jnp.dot(x_ref[...] * norm, w_ref[...],
                             preferred_element_type=jnp.float32)

    return pl.pallas_call(
        body,
        grid=(N // BN,),
        in_specs=[
            pl.BlockSpec((BN, F_IN), lambda i: (i, 0)),
            pl.BlockSpec((1, BN, 1), lambda i: (0, i, 0)),
            pl.BlockSpec((1, BN, 1), lambda i: (1, i, 0)),
            pl.BlockSpec((F_IN, F_HID), lambda i: (0, 0)),
        ],
        out_specs=[
            pl.BlockSpec((BN, F_HID), lambda i: (i, 0)),
            pl.BlockSpec((BN, 1), lambda i: (i, 0)),
        ],
        out_shape=[
            jax.ShapeDtypeStruct((N, F_HID), jnp.float32),
            jax.ShapeDtypeStruct((N, 1), jnp.float32),
        ],
    )(x, deg2, deg2, W0)


def _tc_mid(agg, norm, bias, W, Fo):
    def body(aa_ref, ab_ref, n_ref, b_ref, w_ref, o_ref):
        a = jnp.concatenate([aa_ref[0], ab_ref[0]], axis=1)
        nv = n_ref[...]
        h = jnp.maximum(a * nv + b_ref[...], 0.0)
        o_ref[...] = jnp.dot(h * nv, w_ref[...],
                             preferred_element_type=jnp.float32)

    return pl.pallas_call(
        body,
        grid=(N // BN,),
        in_specs=[
            pl.BlockSpec((1, BN, 64), lambda i: (0, i, 0)),
            pl.BlockSpec((1, BN, 64), lambda i: (1, i, 0)),
            pl.BlockSpec((BN, 1), lambda i: (i, 0)),
            pl.BlockSpec((1, F_HID), lambda i: (0, 0)),
            pl.BlockSpec((F_HID, Fo), lambda i: (0, 0)),
        ],
        out_specs=pl.BlockSpec((BN, Fo), lambda i: (i, 0)),
        out_shape=jax.ShapeDtypeStruct((N, Fo), jnp.float32),
    )(agg, agg, norm, bias, W)


def _tc_final(agg, norm, bias):
    def body(aa_ref, ab_ref, n_ref, b_ref, o_ref):
        o_ref[...] = (aa_ref[0] + ab_ref[0]) * n_ref[...] + b_ref[...]

    return pl.pallas_call(
        body,
        grid=(N // BN,),
        in_specs=[
            pl.BlockSpec((1, BN, F_OUT), lambda i: (0, i, 0)),
            pl.BlockSpec((1, BN, F_OUT), lambda i: (1, i, 0)),
            pl.BlockSpec((BN, 1), lambda i: (i, 0)),
            pl.BlockSpec((1, F_OUT), lambda i: (0, 0)),
        ],
        out_specs=pl.BlockSpec((BN, F_OUT), lambda i: (i, 0)),
        out_shape=jax.ShapeDtypeStruct((N, F_OUT), jnp.float32),
    )(agg, agg, norm, bias)


def kernel(x, edge_index, W0, b0, W1, b1, W2, b2):
    src = edge_index[0]
    dst = edge_index[1]
    pad = E_PAD - E
    srcp = jnp.concatenate([src, jnp.zeros((pad,), jnp.int32)])
    dstp = jnp.concatenate([dst, jnp.full((pad,), N, jnp.int32)])
    src3 = srcp.reshape(NW, K, CHUNK)
    dst3 = dstp.reshape(NW, K, CHUNK)
    src2 = srcp * 2
    srcs = jnp.stack([src2, src2 + 1]).reshape(NC, NS, K2, CHUNK)
    dst4 = dstp.reshape(NS, K2, CHUNK)

    deg = _deg_kernel(dst3)
    deg2 = deg.reshape(NC, NPAD, 1)
    p0, norm = _tc1(x, deg2, W0)
    agg0 = _agg128(p0.reshape(2 * N, 64), srcs, dst4)
    p1 = _tc_mid(agg0, norm, b0.reshape(1, -1), W1, F_HID)
    agg1 = _agg128(p1.reshape(2 * N, 64), srcs, dst4)
    p2 = _tc_mid(agg1, norm, b1.reshape(1, -1), W2, F_OUT)
    agg2 = _agg64(p2, src3, dst3)
    return _tc_final(agg2, norm, b2.reshape(1, -1))


# speedup vs baseline: 1.0132x; 1.0132x over previous
import functools

import jax
import jax.numpy as jnp
from jax import lax
from jax.experimental import pallas as pl
from jax.experimental.pallas import tpu as pltpu
from jax.experimental.pallas import tpu_sc as plsc

N = 10000
F_IN = 128
F_HID = 128
F_OUT = 64
E = 320000

NC = 2
NS = 16
NW = NC * NS
CHUNK = 128
K = 80
K2 = 160
E_PAD = NW * K * CHUNK
NPAD = 10240
STRIPE = NPAD // NS
NBUF = 4

_mesh = plsc.VectorSubcoreMesh(core_axis_name="c", subcore_axis_name="s")
_sc_params = pltpu.CompilerParams(use_tc_tiling_on_sc=False)


@functools.partial(
    pl.kernel,
    out_type=jax.ShapeDtypeStruct((NC, NPAD), jnp.float32),
    mesh=_mesh,
    compiler_params=_sc_params,
    scratch_types=[
        pltpu.VMEM((K, CHUNK), jnp.int32),
        pltpu.VMEM((CHUNK,), jnp.float32),
        pltpu.VMEM((STRIPE,), jnp.float32),
        pltpu.VMEM_SHARED((NPAD,), jnp.float32),
        pltpu.SemaphoreType.DMA,
    ],
)
def _deg_kernel(dst_hbm, deg_out, idx_v, ones_v, z_v, deg_sh, sem):
    c = lax.axis_index("c")
    s = lax.axis_index("s")
    w = s * NC + c

    zero16 = jnp.zeros((16,), jnp.float32)
    one16 = jnp.ones((16,), jnp.float32)

    def fill_z(i, carry):
        z_v[pl.ds(i * 16, 16)] = zero16
        return carry

    lax.fori_loop(0, STRIPE // 16, fill_z, 0)

    def fill_o(i, carry):
        ones_v[pl.ds(i * 16, 16)] = one16
        return carry

    lax.fori_loop(0, CHUNK // 16, fill_o, 0)

    pltpu.sync_copy(z_v, deg_sh.at[pl.ds(s * STRIPE, STRIPE)])
    pltpu.sync_copy(dst_hbm.at[w], idx_v)
    plsc.subcore_barrier()

    GRP = 8

    def group(g, carry):
        for b in range(GRP):
            pltpu.async_copy(
                ones_v, deg_sh.at[idx_v.at[g * GRP + b]], sem, add=True)
        for b in range(GRP):
            pltpu.make_async_copy(
                ones_v, deg_sh.at[idx_v.at[g * GRP + b]], sem).wait()
        return carry

    lax.fori_loop(0, K // GRP, group, 0)
    plsc.subcore_barrier()
    pltpu.sync_copy(deg_sh.at[pl.ds(s * STRIPE, STRIPE)],
                    deg_out.at[c, pl.ds(s * STRIPE, STRIPE)])


AGG_NBUF = 5


def _make_agg(F, colsplit, KC):
    scratch = [
        pltpu.VMEM((KC, CHUNK), jnp.int32),
        pltpu.VMEM((KC, CHUNK), jnp.int32),
        pltpu.VMEM((AGG_NBUF, CHUNK, F), jnp.float32),
        pltpu.VMEM_SHARED((NPAD, F), jnp.float32),
    ] + [pltpu.SemaphoreType.DMA] * (AGG_NBUF + 1)

    @functools.partial(
        pl.kernel,
        out_type=jax.ShapeDtypeStruct((NC, NPAD, F), jnp.float32),
        mesh=_mesh,
        compiler_params=_sc_params,
        scratch_types=scratch,
    )
    def _agg(p_hbm, src_hbm, dst_hbm, out, si_v, di_v, bufs, agg_sh, *sems):
        gsems, ssem = sems[:AGG_NBUF], sems[AGG_NBUF]
        c = lax.axis_index("c")
        s = lax.axis_index("s")

        zero16 = jnp.zeros((16,), jnp.float32)

        def zrow(i, carry):
            for j in range(F // 16):
                bufs[0, i, pl.ds(j * 16, 16)] = zero16
            return carry

        lax.fori_loop(0, CHUNK, zrow, 0)
        for i in range(STRIPE // CHUNK):
            pltpu.sync_copy(
                bufs.at[0], agg_sh.at[pl.ds(s * STRIPE + i * CHUNK, CHUNK)])
        if colsplit:
            pltpu.sync_copy(src_hbm.at[c, s], si_v)
            pltpu.sync_copy(dst_hbm.at[s], di_v)
        else:
            w = s * NC + c
            pltpu.sync_copy(src_hbm.at[w], si_v)
            pltpu.sync_copy(dst_hbm.at[w], di_v)
        plsc.subcore_barrier()

        def gather(jj, b):
            pltpu.async_copy(p_hbm.at[si_v.at[jj]], bufs.at[b], gsems[b])

        def gather_wait(jj, b):
            pltpu.make_async_copy(
                p_hbm.at[si_v.at[jj]], bufs.at[b], gsems[b]).wait()

        for b in range(AGG_NBUF - 1):
            gather(b, b)

        def step(g, carry):
            for i in range(AGG_NBUF):
                jj = g * AGG_NBUF + i
                gather_wait(jj, i)
                pltpu.async_copy(
                    bufs.at[i], agg_sh.at[di_v.at[jj]], ssem, add=True).wait()

                @pl.when(jj + AGG_NBUF - 1 < KC)
                def _():
                    gather(jj + AGG_NBUF - 1, (i + AGG_NBUF - 1) % AGG_NBUF)
            return carry

        lax.fori_loop(0, KC // AGG_NBUF, step, 0)
        plsc.subcore_barrier()
        pltpu.sync_copy(agg_sh.at[pl.ds(s * STRIPE, STRIPE)],
                        out.at[c, pl.ds(s * STRIPE, STRIPE)])

    return _agg


_agg128 = _make_agg(64, True, K2)
_agg64 = _make_agg(F_OUT, False, K)


BN = 1000


def _tc1(x, deg2, W0):
    def body(x_ref, da_ref, db_ref, w_ref, p_ref, n_ref):
        deg = da_ref[0] + db_ref[0]
        norm = jnp.where(deg > 0, lax.rsqrt(jnp.maximum(deg, 1.0)), 0.0)
        n_ref[...] = norm
        p_ref[...] = jnp.dot(x_ref[...] * norm, w_ref[...],
                             preferred_element_type=jnp.float32)

    return pl.pallas_call(
        body,
        grid=(N // BN,),
        in_specs=[
            pl.BlockSpec((BN, F_IN), lambda i: (i, 0)),
            pl.BlockSpec((1, BN, 1), lambda i: (0, i, 0)),
            pl.BlockSpec((1, BN, 1), lambda i: (1, i, 0)),
            pl.BlockSpec((F_IN, F_HID), lambda i: (0, 0)),
        ],
        out_specs=[
            pl.BlockSpec((BN, F_HID), lambda i: (i, 0)),
            pl.BlockSpec((BN, 1), lambda i: (i, 0)),
        ],
        out_shape=[
            jax.ShapeDtypeStruct((N, F_HID), jnp.float32),
            jax.ShapeDtypeStruct((N, 1), jnp.float32),
        ],
    )(x, deg2, deg2, W0)


def _tc_mid(agg, norm, bias, W, Fo):
    def body(aa_ref, ab_ref, n_ref, b_ref, w_ref, o_ref):
        a = jnp.concatenate([aa_ref[0], ab_ref[0]], axis=1)
        nv = n_ref[...]
        h = jnp.maximum(a * nv + b_ref[...], 0.0)
        o_ref[...] = jnp.dot(h * nv, w_ref[...],
                             preferred_element_type=jnp.float32)

    return pl.pallas_call(
        body,
        grid=(N // BN,),
        in_specs=[
            pl.BlockSpec((1, BN, 64), lambda i: (0, i, 0)),
            pl.BlockSpec((1, BN, 64), lambda i: (1, i, 0)),
            pl.BlockSpec((BN, 1), lambda i: (i, 0)),
            pl.BlockSpec((1, F_HID), lambda i: (0, 0)),
            pl.BlockSpec((F_HID, Fo), lambda i: (0, 0)),
        ],
        out_specs=pl.BlockSpec((BN, Fo), lambda i: (i, 0)),
        out_shape=jax.ShapeDtypeStruct((N, Fo), jnp.float32),
    )(agg, agg, norm, bias, W)


def _tc_final(agg, norm, bias):
    def body(aa_ref, ab_ref, n_ref, b_ref, o_ref):
        o_ref[...] = (aa_ref[0] + ab_ref[0]) * n_ref[...] + b_ref[...]

    return pl.pallas_call(
        body,
        grid=(N // BN,),
        in_specs=[
            pl.BlockSpec((1, BN, F_OUT), lambda i: (0, i, 0)),
            pl.BlockSpec((1, BN, F_OUT), lambda i: (1, i, 0)),
            pl.BlockSpec((BN, 1), lambda i: (i, 0)),
            pl.BlockSpec((1, F_OUT), lambda i: (0, 0)),
        ],
        out_specs=pl.BlockSpec((BN, F_OUT), lambda i: (i, 0)),
        out_shape=jax.ShapeDtypeStruct((N, F_OUT), jnp.float32),
    )(agg, agg, norm, bias)


def kernel(x, edge_index, W0, b0, W1, b1, W2, b2):
    src = edge_index[0]
    dst = edge_index[1]
    pad = E_PAD - E
    srcp = jnp.concatenate([src, jnp.zeros((pad,), jnp.int32)])
    dstp = jnp.concatenate([dst, jnp.full((pad,), N, jnp.int32)])
    src3 = srcp.reshape(NW, K, CHUNK)
    dst3 = dstp.reshape(NW, K, CHUNK)
    src2 = srcp * 2
    srcs = jnp.stack([src2, src2 + 1]).reshape(NC, NS, K2, CHUNK)
    dst4 = dstp.reshape(NS, K2, CHUNK)

    deg = _deg_kernel(dst3)
    deg2 = deg.reshape(NC, NPAD, 1)
    p0, norm = _tc1(x, deg2, W0)
    agg0 = _agg128(p0.reshape(2 * N, 64), srcs, dst4)
    p1 = _tc_mid(agg0, norm, b0.reshape(1, -1), W1, F_HID)
    agg1 = _agg128(p1.reshape(2 * N, 64), srcs, dst4)
    p2 = _tc_mid(agg1, norm, b1.reshape(1, -1), W2, F_OUT)
    agg2 = _agg64(p2, src3, dst3)
    return _tc_final(agg2, norm, b2.reshape(1, -1))


# speedup vs baseline: 1.0237x; 1.0104x over previous
import functools

import jax
import jax.numpy as jnp
from jax import lax
from jax.experimental import pallas as pl
from jax.experimental.pallas import tpu as pltpu
from jax.experimental.pallas import tpu_sc as plsc

N = 10000
F_IN = 128
F_HID = 128
F_OUT = 64
E = 320000

NC = 2
NS = 16
NW = NC * NS
CHUNK = 128
K = 80
K2 = 160
E_PAD = NW * K * CHUNK
NPAD = 10240
STRIPE = NPAD // NS
NBUF = 4

_mesh = plsc.VectorSubcoreMesh(core_axis_name="c", subcore_axis_name="s")
_sc_params = pltpu.CompilerParams(use_tc_tiling_on_sc=False)


@functools.partial(
    pl.kernel,
    out_type=jax.ShapeDtypeStruct((NC, NPAD), jnp.float32),
    mesh=_mesh,
    compiler_params=_sc_params,
    scratch_types=[
        pltpu.VMEM((K, CHUNK), jnp.int32),
        pltpu.VMEM((CHUNK,), jnp.float32),
        pltpu.VMEM((STRIPE,), jnp.float32),
        pltpu.VMEM_SHARED((NPAD,), jnp.float32),
        pltpu.SemaphoreType.DMA,
    ],
)
def _deg_kernel(dst_hbm, deg_out, idx_v, ones_v, z_v, deg_sh, sem):
    c = lax.axis_index("c")
    s = lax.axis_index("s")
    w = s * NC + c

    zero16 = jnp.zeros((16,), jnp.float32)
    one16 = jnp.ones((16,), jnp.float32)

    def fill_z(i, carry):
        z_v[pl.ds(i * 16, 16)] = zero16
        return carry

    lax.fori_loop(0, STRIPE // 16, fill_z, 0)

    def fill_o(i, carry):
        ones_v[pl.ds(i * 16, 16)] = one16
        return carry

    lax.fori_loop(0, CHUNK // 16, fill_o, 0)

    pltpu.sync_copy(z_v, deg_sh.at[pl.ds(s * STRIPE, STRIPE)])
    pltpu.sync_copy(dst_hbm.at[w], idx_v)
    plsc.subcore_barrier()

    GRP = 8

    def group(g, carry):
        for b in range(GRP):
            pltpu.async_copy(
                ones_v, deg_sh.at[idx_v.at[g * GRP + b]], sem, add=True)
        for b in range(GRP):
            pltpu.make_async_copy(
                ones_v, deg_sh.at[idx_v.at[g * GRP + b]], sem).wait()
        return carry

    lax.fori_loop(0, K // GRP, group, 0)
    plsc.subcore_barrier()
    pltpu.sync_copy(deg_sh.at[pl.ds(s * STRIPE, STRIPE)],
                    deg_out.at[c, pl.ds(s * STRIPE, STRIPE)])


AGG_NBUF = 5


def _make_agg(F, colsplit, KC):
    scratch = [
        pltpu.VMEM((KC, CHUNK), jnp.int32),
        pltpu.VMEM((KC, CHUNK), jnp.int32),
        pltpu.VMEM((AGG_NBUF, CHUNK, F), jnp.float32),
        pltpu.VMEM_SHARED((NPAD, F), jnp.float32),
    ] + [pltpu.SemaphoreType.DMA] * (AGG_NBUF + 1)

    @functools.partial(
        pl.kernel,
        out_type=jax.ShapeDtypeStruct((NC, NPAD, F), jnp.float32),
        mesh=_mesh,
        compiler_params=_sc_params,
        scratch_types=scratch,
    )
    def _agg(p_hbm, src_hbm, dst_hbm, out, si_v, di_v, bufs, agg_sh, *sems):
        gsems, ssem = sems[:AGG_NBUF], sems[AGG_NBUF]
        c = lax.axis_index("c")
        s = lax.axis_index("s")

        zero16 = jnp.zeros((16,), jnp.float32)

        def zrow(i, carry):
            for j in range(F // 16):
                bufs[0, i, pl.ds(j * 16, 16)] = zero16
            return carry

        lax.fori_loop(0, CHUNK, zrow, 0)
        for i in range(STRIPE // CHUNK):
            pltpu.sync_copy(
                bufs.at[0], agg_sh.at[pl.ds(s * STRIPE + i * CHUNK, CHUNK)])
        if colsplit:
            pltpu.sync_copy(src_hbm.at[c, s], si_v)
            pltpu.sync_copy(dst_hbm.at[s], di_v)
        else:
            w = s * NC + c
            pltpu.sync_copy(src_hbm.at[w], si_v)
            pltpu.sync_copy(dst_hbm.at[w], di_v)
        plsc.subcore_barrier()

        def gather(jj, b):
            pltpu.async_copy(p_hbm.at[si_v.at[jj]], bufs.at[b], gsems[b])

        def gather_wait(jj, b):
            pltpu.make_async_copy(
                p_hbm.at[si_v.at[jj]], bufs.at[b], gsems[b]).wait()

        for b in range(AGG_NBUF - 1):
            gather(b, b)

        def step(g, carry):
            for i in range(AGG_NBUF):
                jj = g * AGG_NBUF + i
                gather_wait(jj, i)
                pltpu.async_copy(
                    bufs.at[i], agg_sh.at[di_v.at[jj]], ssem, add=True).wait()

                @pl.when(jj + AGG_NBUF - 1 < KC)
                def _():
                    gather(jj + AGG_NBUF - 1, (i + AGG_NBUF - 1) % AGG_NBUF)
            return carry

        lax.fori_loop(0, KC // AGG_NBUF, step, 0)
        plsc.subcore_barrier()
        pltpu.sync_copy(agg_sh.at[pl.ds(s * STRIPE, STRIPE)],
                        out.at[c, pl.ds(s * STRIPE, STRIPE)])

    return _agg


_agg128 = _make_agg(64, True, K2)
_agg64 = _make_agg(F_OUT, False, K)


BN = 1000


def _tc1(x, deg2, W0):
    def body(x_ref, da_ref, db_ref, w_ref, p_ref, n_ref):
        deg = da_ref[0] + db_ref[0]
        norm = jnp.where(deg > 0, lax.rsqrt(jnp.maximum(deg, 1.0)), 0.0)
        n_ref[...] = norm
        p_ref[...] = jnp.dot(x_ref[...] * norm, w_ref[...],
                             preferred_element_type=jnp.float32)

    return pl.pallas_call(
        body,
        grid=(N // BN,),
        in_specs=[
            pl.BlockSpec((BN, F_IN), lambda i: (i, 0)),
            pl.BlockSpec((1, BN, 1), lambda i: (0, i, 0)),
            pl.BlockSpec((1, BN, 1), lambda i: (1, i, 0)),
            pl.BlockSpec((F_IN, F_HID), lambda i: (0, 0)),
        ],
        out_specs=[
            pl.BlockSpec((BN, F_HID), lambda i: (i, 0)),
            pl.BlockSpec((BN, 1), lambda i: (i, 0)),
        ],
        out_shape=[
            jax.ShapeDtypeStruct((N, F_HID), jnp.float32),
            jax.ShapeDtypeStruct((N, 1), jnp.float32),
        ],
    )(x, deg2, deg2, W0)


def _tc_mid(agg, norm, bias, W, Fo):
    def body(aa_ref, ab_ref, n_ref, b_ref, w_ref, o_ref):
        a = jnp.concatenate([aa_ref[0], ab_ref[0]], axis=1)
        nv = n_ref[...]
        h = jnp.maximum(a * nv + b_ref[...], 0.0)
        o_ref[...] = jnp.dot(h * nv, w_ref[...],
                             preferred_element_type=jnp.float32)

    return pl.pallas_call(
        body,
        grid=(N // BN,),
        in_specs=[
            pl.BlockSpec((1, BN, 64), lambda i: (0, i, 0)),
            pl.BlockSpec((1, BN, 64), lambda i: (1, i, 0)),
            pl.BlockSpec((BN, 1), lambda i: (i, 0)),
            pl.BlockSpec((1, F_HID), lambda i: (0, 0)),
            pl.BlockSpec((F_HID, Fo), lambda i: (0, 0)),
        ],
        out_specs=pl.BlockSpec((BN, Fo), lambda i: (i, 0)),
        out_shape=jax.ShapeDtypeStruct((N, Fo), jnp.float32),
    )(agg, agg, norm, bias, W)


def _tc_final(agg, norm, bias):
    def body(aa_ref, ab_ref, n_ref, b_ref, o_ref):
        o_ref[...] = (aa_ref[0] + ab_ref[0]) * n_ref[...] + b_ref[...]

    return pl.pallas_call(
        body,
        grid=(N // BN,),
        in_specs=[
            pl.BlockSpec((1, BN, F_OUT), lambda i: (0, i, 0)),
            pl.BlockSpec((1, BN, F_OUT), lambda i: (1, i, 0)),
            pl.BlockSpec((BN, 1), lambda i: (i, 0)),
            pl.BlockSpec((1, F_OUT), lambda i: (0, 0)),
        ],
        out_specs=pl.BlockSpec((BN, F_OUT), lambda i: (i, 0)),
        out_shape=jax.ShapeDtypeStruct((N, F_OUT), jnp.float32),
    )(agg, agg, norm, bias)


def kernel(x, edge_index, W0, b0, W1, b1, W2, b2):
    src = edge_index[0]
    dst = edge_index[1]
    pad = E_PAD - E
    srcp = jnp.concatenate([src, jnp.zeros((pad,), jnp.int32)])
    dstp = jnp.concatenate(
        [dst, N + (jnp.arange(pad, dtype=jnp.int32) % (NPAD - N))])
    src3 = srcp.reshape(NW, K, CHUNK)
    dst3 = dstp.reshape(NW, K, CHUNK)
    src2 = srcp * 2
    srcs = jnp.stack([src2, src2 + 1]).reshape(NC, NS, K2, CHUNK)
    dst4 = dstp.reshape(NS, K2, CHUNK)

    deg = _deg_kernel(dst3)
    deg2 = deg.reshape(NC, NPAD, 1)
    p0, norm = _tc1(x, deg2, W0)
    agg0 = _agg128(p0.reshape(2 * N, 64), srcs, dst4)
    p1 = _tc_mid(agg0, norm, b0.reshape(1, -1), W1, F_HID)
    agg1 = _agg128(p1.reshape(2 * N, 64), srcs, dst4)
    p2 = _tc_mid(agg1, norm, b1.reshape(1, -1), W2, F_OUT)
    agg2 = _agg64(p2, src3, dst3)
    return _tc_final(agg2, norm, b2.reshape(1, -1))
